# baseline (speedup 1.0000x reference)
import functools

import jax
import jax.numpy as jnp
from jax import lax
from jax.experimental import pallas as pl
from jax.experimental.pallas import tpu as pltpu
from jax.experimental.pallas import tpu_sc as plsc

_B, _P, _C = 16384, 1000, 4
_NW = 32
_BW = _B // _NW
_PPS = 8
_NV = _BW // 16


def _sc_call(xt):
    mesh = plsc.VectorSubcoreMesh(core_axis_name="c", subcore_axis_name="s")

    @functools.partial(
        pl.kernel,
        mesh=mesh,
        out_type=jax.ShapeDtypeStruct((_P, _C, _B), jnp.float32),
        scratch_types=[
            pltpu.VMEM((_PPS, _BW), jnp.float32),
            pltpu.VMEM((_PPS, _C, _BW), jnp.float32),
        ],
    )
    def k(xt_hbm, out_hbm, x_v, o_v):
        wid = lax.axis_index("s") * 2 + lax.axis_index("c")
        b0 = wid * _BW

        def stage(t, carry):
            p0 = t * _PPS
            pltpu.sync_copy(xt_hbm.at[pl.ds(p0, _PPS), pl.ds(b0, _BW)], x_v)

            def row(i, carry2):
                def col(j, carry3):
                    xv = x_v[i, pl.ds(j * 16, 16)]
                    o_v[i, 0, pl.ds(j * 16, 16)] = jnp.where(xv == 1.0, 1.0, 0.0)
                    o_v[i, 1, pl.ds(j * 16, 16)] = jnp.where(xv == 2.0, 1.0, 0.0)
                    o_v[i, 2, pl.ds(j * 16, 16)] = jnp.where(xv == 3.0, 1.0, 0.0)
                    o_v[i, 3, pl.ds(j * 16, 16)] = jnp.where(xv == 4.0, 1.0, 0.0)
                    return carry3

                return lax.fori_loop(0, _NV, col, carry2)

            lax.fori_loop(0, _PPS, row, carry)
            pltpu.sync_copy(
                o_v, out_hbm.at[pl.ds(p0, _PPS), :, pl.ds(b0, _BW)])
            return carry

        lax.fori_loop(0, _P // _PPS, stage, 0)

    return k(xt)


def kernel(x):
    xt = x.T
    out_t = _sc_call(xt)
    return out_t.transpose(2, 0, 1)


# speedup vs baseline: 4.1046x; 4.1046x over previous
import jax
import jax.numpy as jnp
from jax.experimental import pallas as pl

_B, _P, _C = 16384, 1000, 4
_BB = 1024


def _onehot_body(xt_ref, o_ref):
    xt = xt_ref[...]
    for v in (1, 2, 3, 4):
        o_ref[:, v - 1, :] = (xt == jnp.float32(v)).astype(jnp.float32)


def kernel(x):
    xt = x.T
    out_t = pl.pallas_call(
        _onehot_body,
        grid=(_B // _BB,),
        in_specs=[pl.BlockSpec((_P, _BB), lambda i: (0, i))],
        out_specs=pl.BlockSpec((_P, _C, _BB), lambda i: (0, 0, i)),
        out_shape=jax.ShapeDtypeStruct((_P, _C, _B), jnp.float32),
    )(xt)
    return out_t.transpose(2, 0, 1)
